# baseline (speedup 1.0000x reference)
import functools
import jax
import jax.numpy as jnp
from jax import lax
from jax.experimental import pallas as pl
from jax.experimental.pallas import tpu as pltpu
from jax.experimental.pallas import tpu_sc as plsc

N = 10000
E = 320000
F = 128
H = 128
C = 64

NC = 2
NS = 16
NW = NC * NS
K = 80
NCHUNK = 126
EPW = K * NCHUNK
EP = NW * EPW
NP = 10240
RPT = NP // NS
DK = 80
TRASH = NP - 1


def _zero_fill(ref, nrow, ncol):
    z16 = jnp.zeros((16,), jnp.float32)

    def _f(i, c):
        for j in range(ncol // 16):
            ref[i, pl.ds(j * 16, 16)] = z16
        return c
    lax.fori_loop(0, nrow, _f, 0)


def _iota_fill(idx_buf, base, n):
    lanes = lax.iota(jnp.int32, 16)
    for q in range(n // 16):
        idx_buf[pl.ds(q * 16, 16)] = base + q * 16 + lanes


def _sc_agg_body(adj_ref, hw_ref, agg_out,
                 src2d, dst_a, dst_b, rows_a, rows_b,
                 semg_a, semg_b, semi_a, semi_b, agg_sh):
    cid = lax.axis_index("c")
    sid = lax.axis_index("s")
    wid = sid * NC + cid

    _zero_fill(rows_a, K, F)

    for r in range(RPT // DK):
        _iota_fill(dst_a, sid * RPT + r * DK, DK)
        pltpu.sync_copy(rows_a, agg_sh.at[dst_a])

    pltpu.sync_copy(adj_ref.at[0, wid], src2d)
    plsc.subcore_barrier()

    pltpu.async_copy(adj_ref.at[1, wid, 0], dst_a, semi_a)
    pltpu.async_copy(adj_ref.at[1, wid, 1], dst_b, semi_b)
    pltpu.async_copy(hw_ref.at[src2d.at[0]], rows_a, semg_a)
    pltpu.async_copy(hw_ref.at[src2d.at[1]], rows_b, semg_b)

    def _step(j, dst_x, rows_x, semg_x, semi_x):
        pltpu.make_async_copy(hw_ref.at[src2d.at[j]], rows_x, semg_x).wait()
        pltpu.make_async_copy(adj_ref.at[1, wid, j], dst_x, semi_x).wait()
        pltpu.sync_copy(rows_x, agg_sh.at[dst_x], add=True)

        @pl.when(j + 2 < NCHUNK)
        def _():
            pltpu.async_copy(adj_ref.at[1, wid, j + 2], dst_x, semi_x)
            pltpu.async_copy(hw_ref.at[src2d.at[j + 2]], rows_x, semg_x)

    def _pair(m, c):
        _step(2 * m, dst_a, rows_a, semg_a, semi_a)
        _step(2 * m + 1, dst_b, rows_b, semg_b, semi_b)
        return c
    lax.fori_loop(0, NCHUNK // 2, _pair, 0)

    plsc.subcore_barrier()

    for r in range(RPT // DK):
        base = sid * RPT + r * DK
        _iota_fill(dst_a, base, DK)
        pltpu.async_copy(agg_sh.at[dst_a], rows_a, semg_a).wait()
        pltpu.sync_copy(rows_a, agg_out.at[cid, pl.ds(base, DK)])


def _sc_agg(adj, hw):
    mesh = plsc.VectorSubcoreMesh(core_axis_name="c", subcore_axis_name="s")
    run = pl.kernel(
        _sc_agg_body,
        out_type=jax.ShapeDtypeStruct((NC, NP, F), jnp.float32),
        mesh=mesh,
        scratch_types=[
            pltpu.VMEM((NCHUNK, K), jnp.int32),
            pltpu.VMEM((K,), jnp.int32),
            pltpu.VMEM((K,), jnp.int32),
            pltpu.VMEM((K, F), jnp.float32),
            pltpu.VMEM((K, F), jnp.float32),
            pltpu.SemaphoreType.DMA,
            pltpu.SemaphoreType.DMA,
            pltpu.SemaphoreType.DMA,
            pltpu.SemaphoreType.DMA,
            pltpu.VMEM_SHARED((NP, F), jnp.float32),
        ],
    )
    return run(adj, hw)


def _sc_deg_body(adj_ref, deg0_out, deg1_out,
                 dst2d, ones, rows, sem, deg_sh):
    cid = lax.axis_index("c")
    sid = lax.axis_index("s")
    wid = sid * NC + cid

    o16 = jnp.ones((16,), jnp.float32)

    def _f(i, c):
        for j in range(F // 16):
            ones[i, pl.ds(j * 16, 16)] = o16
        return c
    lax.fori_loop(0, K, _f, 0)
    _zero_fill(rows, DK, F)

    idx80 = dst2d.at[0]
    for deg_out in (deg0_out, deg1_out):
        ell = 0 if deg_out is deg0_out else 1
        for r in range(RPT // DK):
            _iota_fill(idx80, sid * RPT + r * DK, DK)
            pltpu.sync_copy(rows, deg_sh.at[idx80])
        pltpu.sync_copy(adj_ref.at[ell, 1, wid], dst2d)
        plsc.subcore_barrier()

        def _chunk(j, c):
            pltpu.sync_copy(ones, deg_sh.at[dst2d.at[j]], add=True)
            return c
        lax.fori_loop(0, NCHUNK, _chunk, 0)
        plsc.subcore_barrier()

        for r in range(RPT // DK):
            base = sid * RPT + r * DK
            _iota_fill(idx80, base, DK)
            pltpu.async_copy(deg_sh.at[idx80], rows, sem).wait()
            pltpu.sync_copy(rows, deg_out.at[cid, pl.ds(base, DK)])
        _zero_fill(rows, DK, F)


def _sc_deg(adj):
    mesh = plsc.VectorSubcoreMesh(core_axis_name="c", subcore_axis_name="s")
    run = pl.kernel(
        _sc_deg_body,
        out_type=(
            jax.ShapeDtypeStruct((NC, NP, F), jnp.float32),
            jax.ShapeDtypeStruct((NC, NP, F), jnp.float32),
        ),
        mesh=mesh,
        scratch_types=[
            pltpu.VMEM((NCHUNK, K), jnp.int32),
            pltpu.VMEM((K, F), jnp.float32),
            pltpu.VMEM((DK, F), jnp.float32),
            pltpu.SemaphoreType.DMA,
            pltpu.VMEM_SHARED((NP, F), jnp.float32),
        ],
    )
    return run(adj)


def _mm_body(x_ref, w_ref, b_ref, o_ref):
    o_ref[...] = (
        jnp.dot(x_ref[...], w_ref[...], preferred_element_type=jnp.float32)
        + b_ref[...]
    )


def _mm(x, w, b):
    n, f = x.shape
    ho = w.shape[1]
    blk = 1000
    return pl.pallas_call(
        _mm_body,
        grid=(n // blk,),
        in_specs=[
            pl.BlockSpec((blk, f), lambda i: (i, 0)),
            pl.BlockSpec((f, ho), lambda i: (0, 0)),
            pl.BlockSpec((1, ho), lambda i: (0, 0)),
        ],
        out_specs=pl.BlockSpec((blk, ho), lambda i: (i, 0)),
        out_shape=jax.ShapeDtypeStruct((n, ho), jnp.float32),
    )(x, w, b.reshape(1, ho))


def _combine_body(p_ref, d_ref, w_ref, b_ref, o_ref):
    s = p_ref[0] + p_ref[1]
    deg = jnp.maximum(d_ref[0, :, 0:1] + d_ref[1, :, 0:1], 1.0)
    t = jnp.maximum(s / deg, 0.0)
    o_ref[...] = (
        jnp.dot(t, w_ref[...], preferred_element_type=jnp.float32)
        + b_ref[...]
    )


def _combine_mm(p, d, w, b):
    ho = w.shape[1]
    blk = 1024
    return pl.pallas_call(
        _combine_body,
        grid=(NP // blk,),
        in_specs=[
            pl.BlockSpec((NC, blk, F), lambda i: (0, i, 0)),
            pl.BlockSpec((NC, blk, F), lambda i: (0, i, 0)),
            pl.BlockSpec((F, ho), lambda i: (0, 0)),
            pl.BlockSpec((1, ho), lambda i: (0, 0)),
        ],
        out_specs=pl.BlockSpec((blk, ho), lambda i: (i, 0)),
        out_shape=jax.ShapeDtypeStruct((NP, ho), jnp.float32),
    )(p, d, w, b.reshape(1, ho))


def kernel(x, adjs, W0, b0, W1, b1, W_out, b_out):
    npad = EP - E
    pad = jnp.broadcast_to(
        jnp.array([[0], [TRASH]], dtype=adjs.dtype)[None], (2, 2, npad))
    adjs_p = jnp.concatenate([adjs, pad], axis=2)
    adjs_r = adjs_p.reshape(2, 2, NW, NCHUNK, K)

    d0, d1 = _sc_deg(adjs_r)
    hw0 = _mm(x, W0, b0)
    p0 = _sc_agg(adjs_r[0], hw0)
    hw1 = _combine_mm(p0, d0, W1, b1)
    p1 = _sc_agg(adjs_r[1], hw1)
    out = _combine_mm(p1, d1, W_out, b_out)
    return out[:N]
